# Initial kernel scaffold; baseline (speedup 1.0000x reference)
import jax
import jax.numpy as jnp
from jax.experimental import pallas as pl


def kernel(obs, hidden_states, dones, Wi, Wh, bi, bh, Wout, bout):
    raise NotImplementedError("write your pallas kernel here")


# speedup vs baseline: 13.2537x; 13.2537x over previous
import functools

import jax
import jax.numpy as jnp
from jax.experimental import pallas as pl
from jax.experimental.pallas import tpu as pltpu

S, B, D, H, A = 2048, 16, 64, 128, 16
T_BLK = 256
N_BLK = S // T_BLK


def _gru_kernel(x_ref, dones_ref, dnext_ref, h0_ref,
                Wi_ref, Wh_ref, bi_ref, bh_ref, Wout_ref, bout_ref,
                out_ref, h_ref, gi_ref, hs_ref):
    i = pl.program_id(0)

    @pl.when(i == 0)
    def _():
        d0 = dones_ref[0, :].astype(jnp.float32)[:, None]
        h_ref[...] = h0_ref[0] * (1.0 - d0)

    x = x_ref[...].reshape(T_BLK * B, D)
    gi = jnp.dot(x, Wi_ref[...], preferred_element_type=jnp.float32)
    gi_ref[...] = (gi + bi_ref[0]).reshape(T_BLK, B, 3 * H)

    Wh = Wh_ref[...]
    bh = bh_ref[0]

    def step(t, h):
        d_t = dones_ref[t, :].astype(jnp.float32)[:, None]
        h = h * (1.0 - d_t)
        gh = jnp.dot(h, Wh, preferred_element_type=jnp.float32) + bh
        gi_t = gi_ref[t]
        r = jax.nn.sigmoid(gi_t[:, :H] + gh[:, :H])
        z = jax.nn.sigmoid(gi_t[:, H:2 * H] + gh[:, H:2 * H])
        n = jnp.tanh(gi_t[:, 2 * H:] + r * gh[:, 2 * H:])
        h_new = (1.0 - z) * n + z * h
        hs_ref[t] = h_new
        return h_new

    h_final = jax.lax.fori_loop(0, T_BLK, step, h_ref[...])
    h_ref[...] = h_final

    hs = hs_ref[...].reshape(T_BLK * B, H)
    out = jnp.dot(hs, Wout_ref[...], preferred_element_type=jnp.float32)
    out = out + bout_ref[0]

    row = jax.lax.broadcasted_iota(jnp.int32, (T_BLK, B), 0) + i * T_BLK
    split = jnp.where(row == 0, 1, dones_ref[...])
    split_next = dnext_ref[...]
    keep = (1 - split * split_next).astype(jnp.float32)
    out_ref[...] = out.reshape(T_BLK, B, A) * keep[:, :, None]


@jax.jit
def kernel(obs, hidden_states, dones, Wi, Wh, bi, bh, Wout, bout):
    x = obs.reshape(S, B, D)
    d = dones.reshape(S, B)
    d_next = jnp.concatenate([d[1:], jnp.ones((1, B), dtype=d.dtype)], axis=0)

    in_specs = [
            pl.BlockSpec((T_BLK, B, D), lambda i: (i, 0, 0)),
            pl.BlockSpec((T_BLK, B), lambda i: (i, 0)),
            pl.BlockSpec((T_BLK, B), lambda i: (i, 0)),
            pl.BlockSpec((1, B, H), lambda i: (0, 0, 0)),
            pl.BlockSpec((D, 3 * H), lambda i: (0, 0)),
            pl.BlockSpec((H, 3 * H), lambda i: (0, 0)),
            pl.BlockSpec((1, 3 * H), lambda i: (0, 0)),
            pl.BlockSpec((1, 3 * H), lambda i: (0, 0)),
            pl.BlockSpec((H, A), lambda i: (0, 0)),
            pl.BlockSpec((1, A), lambda i: (0, 0)),
        ]

    out = pl.pallas_call(
        _gru_kernel,
        grid=(N_BLK,),
        in_specs=in_specs,
        out_specs=pl.BlockSpec((T_BLK, B, A), lambda i: (i, 0, 0)),
        out_shape=jax.ShapeDtypeStruct((S, B, A), jnp.float32),
        scratch_shapes=[
            pltpu.VMEM((B, H), jnp.float32),
            pltpu.VMEM((T_BLK, B, 3 * H), jnp.float32),
            pltpu.VMEM((T_BLK, B, H), jnp.float32),
        ],
    )(x, d, d_next, hidden_states,
      Wi, Wh, bi.reshape(1, 3 * H), bh.reshape(1, 3 * H),
      Wout, bout.reshape(1, A))
    return out.reshape(S * B, A)


# speedup vs baseline: 18.8991x; 1.4259x over previous
import functools

import jax
import jax.numpy as jnp
from jax.experimental import pallas as pl
from jax.experimental.pallas import tpu as pltpu

S, B, D, H, A = 2048, 16, 64, 128, 16
T_BLK = 256
N_BLK = S // T_BLK


def _gru_kernel(x_ref, dones_ref, dnext_ref, h0_ref,
                Wi_ref, Wh_ref, bi_ref, bh_ref, Wout_ref, bout_ref,
                out_ref, h_ref, gi_ref, hs_ref):
    i = pl.program_id(0)

    @pl.when(i == 0)
    def _():
        d0 = dones_ref[0, :].astype(jnp.float32)[:, None]
        h_ref[...] = h0_ref[0] * (1.0 - d0)

    x = x_ref[...].reshape(T_BLK * B, D)
    gi = jnp.dot(x, Wi_ref[...], preferred_element_type=jnp.float32)
    gi_ref[...] = (gi + bi_ref[0]).reshape(T_BLK, B, 3 * H)

    Wh = Wh_ref[...]
    bh = bh_ref[0]

    def step(t, h):
        m = jnp.dot(h, Wh, preferred_element_type=jnp.float32)
        k = 1.0 - dones_ref[t, :].astype(jnp.float32)[:, None]
        gh = m * k + bh
        h_m = h * k
        gi_t = gi_ref[t]
        r = jax.nn.sigmoid(gi_t[:, :H] + gh[:, :H])
        z = jax.nn.sigmoid(gi_t[:, H:2 * H] + gh[:, H:2 * H])
        n = jnp.tanh(gi_t[:, 2 * H:] + r * gh[:, 2 * H:])
        h_new = (1.0 - z) * n + z * h_m
        hs_ref[t] = h_new
        return h_new

    h_final = jax.lax.fori_loop(0, T_BLK, step, h_ref[...], unroll=8)
    h_ref[...] = h_final

    hs = hs_ref[...].reshape(T_BLK * B, H)
    out = jnp.dot(hs, Wout_ref[...], preferred_element_type=jnp.float32)
    out = out + bout_ref[0]

    row = jax.lax.broadcasted_iota(jnp.int32, (T_BLK, B), 0) + i * T_BLK
    split = jnp.where(row == 0, 1, dones_ref[...])
    split_next = dnext_ref[...]
    keep = (1 - split * split_next).astype(jnp.float32)
    out_ref[...] = out.reshape(T_BLK, B, A) * keep[:, :, None]


@jax.jit
def kernel(obs, hidden_states, dones, Wi, Wh, bi, bh, Wout, bout):
    x = obs.reshape(S, B, D)
    d = dones.reshape(S, B)
    d_next = jnp.concatenate([d[1:], jnp.ones((1, B), dtype=d.dtype)], axis=0)

    in_specs = [
            pl.BlockSpec((T_BLK, B, D), lambda i: (i, 0, 0)),
            pl.BlockSpec((T_BLK, B), lambda i: (i, 0)),
            pl.BlockSpec((T_BLK, B), lambda i: (i, 0)),
            pl.BlockSpec((1, B, H), lambda i: (0, 0, 0)),
            pl.BlockSpec((D, 3 * H), lambda i: (0, 0)),
            pl.BlockSpec((H, 3 * H), lambda i: (0, 0)),
            pl.BlockSpec((1, 3 * H), lambda i: (0, 0)),
            pl.BlockSpec((1, 3 * H), lambda i: (0, 0)),
            pl.BlockSpec((H, A), lambda i: (0, 0)),
            pl.BlockSpec((1, A), lambda i: (0, 0)),
        ]

    out = pl.pallas_call(
        _gru_kernel,
        grid=(N_BLK,),
        in_specs=in_specs,
        out_specs=pl.BlockSpec((T_BLK, B, A), lambda i: (i, 0, 0)),
        out_shape=jax.ShapeDtypeStruct((S, B, A), jnp.float32),
        scratch_shapes=[
            pltpu.VMEM((B, H), jnp.float32),
            pltpu.VMEM((T_BLK, B, 3 * H), jnp.float32),
            pltpu.VMEM((T_BLK, B, H), jnp.float32),
        ],
    )(x, d, d_next, hidden_states,
      Wi, Wh, bi.reshape(1, 3 * H), bh.reshape(1, 3 * H),
      Wout, bout.reshape(1, A))
    return out.reshape(S * B, A)
